# Initial kernel scaffold; baseline (speedup 1.0000x reference)
import jax
import jax.numpy as jnp
from jax.experimental import pallas as pl


def kernel(pov, white_idx, black_idx, p_white_idx, p_black_idx, w_aff_W, w_aff_b, w_fact_W, b_aff_W, b_aff_b, b_fact_W, pw_aff_W, pw_aff_b, pw_fact_W, pb_aff_W, pb_aff_b, pb_fact_W, fc0_W, fc0_b, p_fc0_W, p_fc0_b, p_fc1_W, p_fc1_b, fc1_W, fc1_b, fc2_W, fc2_b, fc3_W, fc3_b):
    raise NotImplementedError("write your pallas kernel here")


# speedup vs baseline: 4.1103x; 4.1103x over previous
import functools

import jax
import jax.numpy as jnp
from jax import lax
from jax.experimental import pallas as pl
from jax.experimental.pallas import tpu as pltpu
from jax.experimental.pallas import tpu_sc as plsc

_D_MAIN = 49152
_D_PAWN = 8192
_F_MAIN = 768
_F_PAWN = 128
_BASE = 160
_P_BASE = 256
_N = 4096

_NC = 2
_NS = 16
_NW = _NC * _NS
_SW = _N // _NW
_KM = 32
_KP = 8
_CS_M = 128 // _KM
_CS_P = 128 // _KP
_NCH_M = _SW // _CS_M
_NCH_P = _SW // _CS_P


def _sc_body(widx, bidx, pwidx, pbidx, wt, bt, pwt, pbt,
             wout, bout, pwout, pbout,
             idx_m, idx_p, rows_m, acc_m, rows_p, acc_p, sem):
    wid = lax.axis_index("s") * _NC + lax.axis_index("c")

    def main_color(idx_hbm, tab_hbm, out_hbm):
        pltpu.sync_copy(idx_hbm.at[pl.ds(wid * _SW * _KM, _SW * _KM)], idx_m)

        def chunk(c, _):
            off = pl.multiple_of(c * 128, 128)
            pltpu.async_copy(tab_hbm.at[idx_m.at[pl.ds(off, 128)]],
                             rows_m, sem).wait()

            def sample(s, _):
                row = c * _CS_M + s
                base = s * _KM
                for v in range(_BASE // 16):
                    sl = pl.ds(v * 16, 16)
                    acc = rows_m[base, sl]
                    for r in range(1, _KM):
                        acc = acc + rows_m[base + r, sl]
                    acc_m[row, sl] = acc
                return 0

            lax.fori_loop(0, _CS_M, sample, 0)
            return ()

        lax.fori_loop(0, _NCH_M, chunk, ())
        pltpu.sync_copy(acc_m, out_hbm.at[pl.ds(wid * _SW, _SW)])

    def pawn_color(idx_hbm, tab_hbm, out_hbm):
        pltpu.sync_copy(idx_hbm.at[pl.ds(wid * _SW * _KP, _SW * _KP)], idx_p)

        def chunk(c, _):
            off = pl.multiple_of(c * 128, 128)
            pltpu.async_copy(tab_hbm.at[idx_p.at[pl.ds(off, 128)]],
                             rows_p, sem).wait()

            def sample(s, _):
                row = c * _CS_P + s
                base = s * _KP
                for v in range(_P_BASE // 16):
                    sl = pl.ds(v * 16, 16)
                    acc = rows_p[base, sl]
                    for r in range(1, _KP):
                        acc = acc + rows_p[base + r, sl]
                    acc_p[row, sl] = acc
                return 0

            lax.fori_loop(0, _CS_P, sample, 0)
            return ()

        lax.fori_loop(0, _NCH_P, chunk, ())
        pltpu.sync_copy(acc_p, out_hbm.at[pl.ds(wid * _SW, _SW)])

    main_color(widx, wt, wout)
    main_color(bidx, bt, bout)
    pawn_color(pwidx, pwt, pwout)
    pawn_color(pbidx, pbt, pbout)


@functools.cache
def _sc_lookup_fn():
    return pl.kernel(
        _sc_body,
        out_type=(
            jax.ShapeDtypeStruct((_N, _BASE), jnp.float32),
            jax.ShapeDtypeStruct((_N, _BASE), jnp.float32),
            jax.ShapeDtypeStruct((_N, _P_BASE), jnp.float32),
            jax.ShapeDtypeStruct((_N, _P_BASE), jnp.float32),
        ),
        mesh=plsc.VectorSubcoreMesh(core_axis_name="c", subcore_axis_name="s",
                                    num_cores=_NC, num_subcores=_NS),
        scratch_types=[
            pltpu.VMEM((_SW * _KM,), jnp.int32),
            pltpu.VMEM((_SW * _KP,), jnp.int32),
            pltpu.VMEM((128, _BASE), jnp.float32),
            pltpu.VMEM((_SW, _BASE), jnp.float32),
            pltpu.VMEM((128, _P_BASE), jnp.float32),
            pltpu.VMEM((_SW, _P_BASE), jnp.float32),
            pltpu.SemaphoreType.DMA,
        ],
        compiler_params=pltpu.CompilerParams(use_tc_tiling_on_sc=False),
    )


_BN = 1024


def _head_body(pov, w, b, pw, pb, wbias, bbias, pwbias, pbbias,
               f0a, f0b, f0c, p0a, p0b, p0c, p1t, p1c, f1t, f1c,
               f2a, f2b, f2c, f3a, f3b, f3c, f3d, out):
    q = pov[...]
    r = 1.0 - q
    wv = w[...] + wbias[...]
    bv = b[...] + bbias[...]
    pwv = pw[...] + pwbias[...]
    pbv = pb[...] + pbbias[...]
    base1 = jnp.maximum(q * wv + r * bv, 0.0)
    base2 = jnp.maximum(q * bv + r * wv, 0.0)
    pb1 = jnp.maximum(q * pwv + r * pbv, 0.0)
    pb2 = jnp.maximum(q * pbv + r * pwv, 0.0)
    dot = functools.partial(jnp.dot, preferred_element_type=jnp.float32)
    p = jnp.maximum(dot(pb1, p0a[...]) + dot(pb2, p0b[...]) + p0c[...], 0.0)
    p = dot(p, p1t[...]) + p1c[...]
    x = jnp.maximum(dot(base1, f0a[...]) + dot(base2, f0b[...]) + f0c[...] + p,
                    0.0)
    y = jnp.maximum(dot(x, f1t[...]) + f1c[...], 0.0)
    z = jnp.maximum(dot(x, f2a[...]) + dot(y, f2b[...]) + f2c[...], 0.0)
    out[...] = dot(x, f3a[...]) + dot(y, f3b[...]) + dot(z, f3c[...]) + f3d[...]


def _row_spec(cols):
    return pl.BlockSpec((_BN, cols), lambda i: (i, 0))


def _full_spec(rows, cols):
    return pl.BlockSpec((rows, cols), lambda i: (0, 0))


def kernel(pov, white_idx, black_idx, p_white_idx, p_black_idx,
           w_aff_W, w_aff_b, w_fact_W, b_aff_W, b_aff_b, b_fact_W,
           pw_aff_W, pw_aff_b, pw_fact_W, pb_aff_W, pb_aff_b, pb_fact_W,
           fc0_W, fc0_b, p_fc0_W, p_fc0_b, p_fc1_W, p_fc1_b,
           fc1_W, fc1_b, fc2_W, fc2_b, fc3_W, fc3_b):
    w_comb = w_aff_W.T + jnp.tile(w_fact_W, (_D_MAIN // _F_MAIN, 1))
    b_comb = b_aff_W.T + jnp.tile(b_fact_W, (_D_MAIN // _F_MAIN, 1))
    pw_comb = pw_aff_W.T + jnp.tile(pw_fact_W, (_D_PAWN // _F_PAWN, 1))
    pb_comb = pb_aff_W.T + jnp.tile(pb_fact_W, (_D_PAWN // _F_PAWN, 1))

    wsum, bsum, pwsum, pbsum = _sc_lookup_fn()(
        white_idx.reshape(-1), black_idx.reshape(-1),
        p_white_idx.reshape(-1), p_black_idx.reshape(-1),
        w_comb, b_comb, pw_comb, pb_comb)

    grid = (_N // _BN,)
    out = pl.pallas_call(
        _head_body,
        grid=grid,
        in_specs=[
            _row_spec(1),
            _row_spec(_BASE), _row_spec(_BASE),
            _row_spec(_P_BASE), _row_spec(_P_BASE),
            _full_spec(1, _BASE), _full_spec(1, _BASE),
            _full_spec(1, _P_BASE), _full_spec(1, _P_BASE),
            _full_spec(_BASE, 16), _full_spec(_BASE, 16), _full_spec(1, 16),
            _full_spec(_P_BASE, 16), _full_spec(_P_BASE, 16), _full_spec(1, 16),
            _full_spec(16, 16), _full_spec(1, 16),
            _full_spec(16, 16), _full_spec(1, 16),
            _full_spec(16, 16), _full_spec(16, 16), _full_spec(1, 16),
            _full_spec(16, 1), _full_spec(16, 1), _full_spec(16, 1),
            _full_spec(1, 1),
        ],
        out_specs=_row_spec(1),
        out_shape=jax.ShapeDtypeStruct((_N, 1), jnp.float32),
    )(
        pov, wsum, bsum, pwsum, pbsum,
        w_aff_b.reshape(1, -1), b_aff_b.reshape(1, -1),
        pw_aff_b.reshape(1, -1), pb_aff_b.reshape(1, -1),
        fc0_W[:, :_BASE].T, fc0_W[:, _BASE:].T, fc0_b.reshape(1, -1),
        p_fc0_W[:, :_P_BASE].T, p_fc0_W[:, _P_BASE:].T, p_fc0_b.reshape(1, -1),
        p_fc1_W.T, p_fc1_b.reshape(1, -1),
        fc1_W.T, fc1_b.reshape(1, -1),
        fc2_W[:, :16].T, fc2_W[:, 16:].T, fc2_b.reshape(1, -1),
        fc3_W[:, :16].T, fc3_W[:, 16:32].T, fc3_W[:, 32:].T,
        fc3_b.reshape(1, -1),
    )
    return out


# speedup vs baseline: 5.7807x; 1.4064x over previous
import functools

import jax
import jax.numpy as jnp
from jax import lax
from jax.experimental import pallas as pl
from jax.experimental.pallas import tpu as pltpu
from jax.experimental.pallas import tpu_sc as plsc

_D_MAIN = 49152
_D_PAWN = 8192
_F_MAIN = 768
_F_PAWN = 128
_BASE = 160
_P_BASE = 256
_N = 4096

_NC = 2
_NS = 16
_NW = _NC * _NS
_SW = _N // _NW
_KM = 32
_KP = 8
_CS_M = 128 // _KM
_CS_P = 128 // _KP
_NCH_M = _SW // _CS_M
_NCH_P = _SW // _CS_P


def _sc_body(widx, bidx, pwidx, pbidx, wt, bt, pwt, pbt,
             wout, bout, pwout, pbout,
             idx_mw, idx_mb, idx_pw, idx_pb, acc_mw, acc_mb, acc_pw, acc_pb,
             sem):
    wid = lax.axis_index("s") * _NC + lax.axis_index("c")
    base = pl.ds(wid * _SW, _SW)

    pltpu.sync_copy(widx.at[:, base], idx_mw)
    pltpu.sync_copy(bidx.at[:, base], idx_mb)
    pltpu.sync_copy(pwidx.at[:, base], idx_pw)
    pltpu.sync_copy(pbidx.at[:, base], idx_pb)

    pltpu.async_copy(wt.at[idx_mw.at[0]], acc_mw, sem).wait()
    pltpu.async_copy(bt.at[idx_mb.at[0]], acc_mb, sem).wait()
    pltpu.async_copy(pwt.at[idx_pw.at[0]], acc_pw, sem).wait()
    pltpu.async_copy(pbt.at[idx_pb.at[0]], acc_pb, sem).wait()

    descs = []
    for j in range(1, _KM):
        descs.append(pltpu.async_copy(wt.at[idx_mw.at[j]], acc_mw, sem,
                                      add=True))
        descs.append(pltpu.async_copy(bt.at[idx_mb.at[j]], acc_mb, sem,
                                      add=True))
    for j in range(1, _KP):
        descs.append(pltpu.async_copy(pwt.at[idx_pw.at[j]], acc_pw, sem,
                                      add=True))
        descs.append(pltpu.async_copy(pbt.at[idx_pb.at[j]], acc_pb, sem,
                                      add=True))
    for d in descs:
        d.wait()

    pltpu.sync_copy(acc_mw, wout.at[base])
    pltpu.sync_copy(acc_mb, bout.at[base])
    pltpu.sync_copy(acc_pw, pwout.at[base])
    pltpu.sync_copy(acc_pb, pbout.at[base])


@functools.cache
def _sc_lookup_fn():
    return pl.kernel(
        _sc_body,
        out_type=(
            jax.ShapeDtypeStruct((_N, _BASE), jnp.float32),
            jax.ShapeDtypeStruct((_N, _BASE), jnp.float32),
            jax.ShapeDtypeStruct((_N, _P_BASE), jnp.float32),
            jax.ShapeDtypeStruct((_N, _P_BASE), jnp.float32),
        ),
        mesh=plsc.VectorSubcoreMesh(core_axis_name="c", subcore_axis_name="s",
                                    num_cores=_NC, num_subcores=_NS),
        scratch_types=[
            pltpu.VMEM((_KM, _SW), jnp.int32),
            pltpu.VMEM((_KM, _SW), jnp.int32),
            pltpu.VMEM((_KP, _SW), jnp.int32),
            pltpu.VMEM((_KP, _SW), jnp.int32),
            pltpu.VMEM((_SW, _BASE), jnp.float32),
            pltpu.VMEM((_SW, _BASE), jnp.float32),
            pltpu.VMEM((_SW, _P_BASE), jnp.float32),
            pltpu.VMEM((_SW, _P_BASE), jnp.float32),
            pltpu.SemaphoreType.DMA,
        ],
        compiler_params=pltpu.CompilerParams(use_tc_tiling_on_sc=False),
    )


_BN = 1024


def _head_body(pov, w, b, pw, pb, wbias, bbias, pwbias, pbbias,
               f0a, f0b, f0c, p0a, p0b, p0c, p1t, p1c, f1t, f1c,
               f2a, f2b, f2c, f3a, f3b, f3c, f3d, out):
    q = pov[...]
    r = 1.0 - q
    wv = w[...] + wbias[...]
    bv = b[...] + bbias[...]
    pwv = pw[...] + pwbias[...]
    pbv = pb[...] + pbbias[...]
    base1 = jnp.maximum(q * wv + r * bv, 0.0)
    base2 = jnp.maximum(q * bv + r * wv, 0.0)
    pb1 = jnp.maximum(q * pwv + r * pbv, 0.0)
    pb2 = jnp.maximum(q * pbv + r * pwv, 0.0)
    dot = functools.partial(jnp.dot, preferred_element_type=jnp.float32)
    p = jnp.maximum(dot(pb1, p0a[...]) + dot(pb2, p0b[...]) + p0c[...], 0.0)
    p = dot(p, p1t[...]) + p1c[...]
    x = jnp.maximum(dot(base1, f0a[...]) + dot(base2, f0b[...]) + f0c[...] + p,
                    0.0)
    y = jnp.maximum(dot(x, f1t[...]) + f1c[...], 0.0)
    z = jnp.maximum(dot(x, f2a[...]) + dot(y, f2b[...]) + f2c[...], 0.0)
    out[...] = dot(x, f3a[...]) + dot(y, f3b[...]) + dot(z, f3c[...]) + f3d[...]


def _row_spec(cols):
    return pl.BlockSpec((_BN, cols), lambda i: (i, 0))


def _full_spec(rows, cols):
    return pl.BlockSpec((rows, cols), lambda i: (0, 0))


def kernel(pov, white_idx, black_idx, p_white_idx, p_black_idx,
           w_aff_W, w_aff_b, w_fact_W, b_aff_W, b_aff_b, b_fact_W,
           pw_aff_W, pw_aff_b, pw_fact_W, pb_aff_W, pb_aff_b, pb_fact_W,
           fc0_W, fc0_b, p_fc0_W, p_fc0_b, p_fc1_W, p_fc1_b,
           fc1_W, fc1_b, fc2_W, fc2_b, fc3_W, fc3_b):
    w_comb = w_aff_W.T + jnp.tile(w_fact_W, (_D_MAIN // _F_MAIN, 1))
    b_comb = b_aff_W.T + jnp.tile(b_fact_W, (_D_MAIN // _F_MAIN, 1))
    pw_comb = pw_aff_W.T + jnp.tile(pw_fact_W, (_D_PAWN // _F_PAWN, 1))
    pb_comb = pb_aff_W.T + jnp.tile(pb_fact_W, (_D_PAWN // _F_PAWN, 1))

    wsum, bsum, pwsum, pbsum = _sc_lookup_fn()(
        white_idx.T, black_idx.T, p_white_idx.T, p_black_idx.T,
        w_comb, b_comb, pw_comb, pb_comb)

    grid = (_N // _BN,)
    out = pl.pallas_call(
        _head_body,
        grid=grid,
        in_specs=[
            _row_spec(1),
            _row_spec(_BASE), _row_spec(_BASE),
            _row_spec(_P_BASE), _row_spec(_P_BASE),
            _full_spec(1, _BASE), _full_spec(1, _BASE),
            _full_spec(1, _P_BASE), _full_spec(1, _P_BASE),
            _full_spec(_BASE, 16), _full_spec(_BASE, 16), _full_spec(1, 16),
            _full_spec(_P_BASE, 16), _full_spec(_P_BASE, 16), _full_spec(1, 16),
            _full_spec(16, 16), _full_spec(1, 16),
            _full_spec(16, 16), _full_spec(1, 16),
            _full_spec(16, 16), _full_spec(16, 16), _full_spec(1, 16),
            _full_spec(16, 1), _full_spec(16, 1), _full_spec(16, 1),
            _full_spec(1, 1),
        ],
        out_specs=_row_spec(1),
        out_shape=jax.ShapeDtypeStruct((_N, 1), jnp.float32),
    )(
        pov, wsum, bsum, pwsum, pbsum,
        w_aff_b.reshape(1, -1), b_aff_b.reshape(1, -1),
        pw_aff_b.reshape(1, -1), pb_aff_b.reshape(1, -1),
        fc0_W[:, :_BASE].T, fc0_W[:, _BASE:].T, fc0_b.reshape(1, -1),
        p_fc0_W[:, :_P_BASE].T, p_fc0_W[:, _P_BASE:].T, p_fc0_b.reshape(1, -1),
        p_fc1_W.T, p_fc1_b.reshape(1, -1),
        fc1_W.T, fc1_b.reshape(1, -1),
        fc2_W[:, :16].T, fc2_W[:, 16:].T, fc2_b.reshape(1, -1),
        fc3_W[:, :16].T, fc3_W[:, 16:32].T, fc3_W[:, 32:].T,
        fc3_b.reshape(1, -1),
    )
    return out
